# baseline (speedup 1.0000x reference)
import functools

import jax
import jax.numpy as jnp
from jax import lax
from jax.experimental import pallas as pl
from jax.experimental.pallas import tpu as pltpu
from jax.experimental.pallas import tpu_sc as plsc

NUM_FEAT = 1000000
HIDDEN_DIM = 32
BATCH = 16384
FIELDS = 26

_INFO = plsc.get_sparse_core_info()
_NC = _INFO.num_cores
_NS = _INFO.num_subcores
_NW = _NC * _NS

_B = BATCH * FIELDS
_BPW = _B // _NW
_NBUF = 3
_CHUNK = 832
_NCHUNK = _BPW // _CHUNK


def _make_kernel():
    mesh = plsc.VectorSubcoreMesh(core_axis_name="c", subcore_axis_name="s")

    @functools.partial(
        pl.kernel,
        mesh=mesh,
        out_type=jax.ShapeDtypeStruct((_B, HIDDEN_DIM), jnp.float32),
        compiler_params=pltpu.CompilerParams(use_tc_tiling_on_sc=False),
        scratch_types=[
            pltpu.VMEM((_BPW,), jnp.int32),
        ] + [pltpu.VMEM((_CHUNK, HIDDEN_DIM), jnp.float32)
             for _ in range(_NBUF)]
          + [pltpu.SemaphoreType.DMA for _ in range(_NBUF + 1)],
    )
    def emb_kernel(idx_hbm, table_hbm, out_hbm, idx_v, *scratch):
        bufs = scratch[:_NBUF]
        isem = scratch[_NBUF]
        gsems = scratch[_NBUF + 1:]
        wid = lax.axis_index("s") * _NC + lax.axis_index("c")
        base = wid * _BPW
        pltpu.async_copy(idx_hbm.at[wid], idx_v, isem).wait()

        def gather(c):
            return pltpu.async_copy(
                table_hbm.at[idx_v.at[pl.ds(c * _CHUNK, _CHUNK)]],
                bufs[c % _NBUF], gsems[c % _NBUF])

        pend = [gather(c) for c in range(_NBUF - 1)]
        for c in range(_NCHUNK):
            if c + _NBUF - 1 < _NCHUNK:
                pend.append(gather(c + _NBUF - 1))
            pend[0].wait()
            pend = pend[1:]
            pltpu.sync_copy(bufs[c % _NBUF],
                            out_hbm.at[pl.ds(base + c * _CHUNK, _CHUNK)])

    return emb_kernel


_EMB = _make_kernel()


@jax.jit
def kernel(x, weight):
    idx = x.astype(jnp.int32).reshape(_NW, _BPW)
    out = _EMB(idx, weight)
    return out.reshape(BATCH, FIELDS, HIDDEN_DIM)
